# Initial kernel scaffold; baseline (speedup 1.0000x reference)
import jax
import jax.numpy as jnp
from jax.experimental import pallas as pl


def kernel(X, edge_index, edge_weight, Wc_i, bc_i, Wl_i, bl_i, Wc_f, bc_f, Wl_f, bl_f, Wc_g, bc_g, Wl_g, bl_g, Wc_o, bc_o, Wl_o, bl_o):
    raise NotImplementedError("write your pallas kernel here")


# speedup vs baseline: 18.8374x; 18.8374x over previous
import jax
import jax.numpy as jnp
from jax import lax
from jax.experimental import pallas as pl
from jax.experimental.pallas import tpu as pltpu
from jax.experimental.pallas import tpu_sc as plsc

N = 10000
D = 128
E = 320000

NC = 2
NS = 16
L = 16
DH = D // NC

N_PAD = 10240
ROWS_W = N_PAD // NS
K = 128
CH = 79
EG = CH * K
NG = NC * NS
E_PAD = NG * EG
VEC_IT = EG // L


def _sc_body(src_hbm, dst_hbm, ew_hbm, x0_hbm, x1_hbm, part_hbm,
             degp, idx_src, idx_dst, ewn, rows, redbuf,
             deg_parts_sh, dis_sh, ax_sh, sem):
    c = lax.axis_index("c")
    s = lax.axis_index("s")

    def _zero(i, _):
        degp[pl.ds(i * L, L)] = jnp.zeros((L,), jnp.float32)
        return 0
    lax.fori_loop(0, N_PAD // L, _zero, 0)

    def _deg_group(g, _):
        pltpu.sync_copy(dst_hbm.at[g], idx_dst)
        pltpu.sync_copy(ew_hbm.at[g], ewn)

        def _dbody(i, _):
            r = i // (K // L)
            k = (i % (K // L)) * L
            di = idx_dst[r, pl.ds(k, L)]
            wv = ewn[r, pl.ds(k, L)]
            plsc.addupdate_scatter(degp, [di], wv)
            return 0
        lax.fori_loop(0, VEC_IT, _dbody, 0)
        return 0
    lax.fori_loop(2 * s, 2 * s + 2, _deg_group, 0)

    pltpu.sync_copy(degp, deg_parts_sh.at[s])
    plsc.subcore_barrier()
    pltpu.sync_copy(deg_parts_sh.at[:, pl.ds(s * ROWS_W, ROWS_W)], redbuf)

    def _red(i, _):
        acc = redbuf[0, pl.ds(i * L, L)]
        for r in range(1, NS):
            acc = acc + redbuf[r, pl.ds(i * L, L)]
        acc = acc + 1.0
        xi = plsc.bitcast(acc, jnp.int32)
        yi = jnp.int32(0x5F3759DF) - lax.shift_right_logical(xi, 1)
        y = plsc.bitcast(yi, jnp.float32)
        for _ in range(3):
            y = y * (1.5 - 0.5 * acc * y * y)
        degp[pl.ds(i * L, L)] = y
        return 0
    lax.fori_loop(0, ROWS_W // L, _red, 0)
    pltpu.sync_copy(degp.at[pl.ds(0, ROWS_W)], dis_sh.at[pl.ds(s * ROWS_W, ROWS_W)])
    plsc.subcore_barrier()

    pltpu.sync_copy(dis_sh, degp)

    def _run_core(xh_hbm):
        def _init_chunk(ch, _):
            base = s * ROWS_W + ch * K
            pltpu.sync_copy(xh_hbm.at[pl.ds(base, K)], rows)

            def _rowblk(kb, _):
                disv = degp[pl.ds(base + kb * L, L)]
                scv = disv * disv
                for k in range(L):
                    sc_v = jnp.broadcast_to(scv[k], (L,))
                    row = kb * L + k
                    for j in range(DH // L):
                        rows[row, pl.ds(j * L, L)] = rows[row, pl.ds(j * L, L)] * sc_v
                return 0
            lax.fori_loop(0, K // L, _rowblk, 0)
            pltpu.sync_copy(rows, ax_sh.at[pl.ds(base, K)])
            return 0
        lax.fori_loop(0, ROWS_W // K, _init_chunk, 0)
        plsc.subcore_barrier()

        def _group(g, _):
            pltpu.sync_copy(src_hbm.at[g], idx_src)
            pltpu.sync_copy(dst_hbm.at[g], idx_dst)
            pltpu.sync_copy(ew_hbm.at[g], ewn)

            def _norm(i, _):
                r = i // (K // L)
                k = (i % (K // L)) * L
                sv = idx_src[r, pl.ds(k, L)]
                dv = idx_dst[r, pl.ds(k, L)]
                w = ewn[r, pl.ds(k, L)]
                ewn[r, pl.ds(k, L)] = (plsc.load_gather(degp, [sv]) * w
                                       * plsc.load_gather(degp, [dv]))
                return 0
            lax.fori_loop(0, VEC_IT, _norm, 0)

            def _chunk(ci, _):
                pltpu.async_copy(xh_hbm.at[idx_src.at[ci]], rows, sem).wait()

                def _edgeblk(kb, _):
                    nvec = ewn[ci, pl.ds(kb * L, L)]
                    for k in range(L):
                        nv = jnp.broadcast_to(nvec[k], (L,))
                        row = kb * L + k
                        for j in range(DH // L):
                            rows[row, pl.ds(j * L, L)] = rows[row, pl.ds(j * L, L)] * nv
                    return 0
                lax.fori_loop(0, K // L, _edgeblk, 0)
                pltpu.sync_copy(rows, ax_sh.at[idx_dst.at[ci]], add=True)
                return 0
            lax.fori_loop(0, CH, _chunk, 0)
            return 0
        lax.fori_loop(2 * s, 2 * s + 2, _group, 0)

    @pl.when(c == 0)
    def _():
        _run_core(x0_hbm)

    @pl.when(c == 1)
    def _():
        _run_core(x1_hbm)

    plsc.subcore_barrier()

    pltpu.sync_copy(ax_sh.at[pl.ds(s * ROWS_W, ROWS_W)],
                    part_hbm.at[c, pl.ds(s * ROWS_W, ROWS_W)])


def _make_sc_kernel():
    mesh = plsc.VectorSubcoreMesh(core_axis_name="c", subcore_axis_name="s",
                                  num_cores=NC, num_subcores=NS)
    return pl.kernel(
        _sc_body,
        out_type=jax.ShapeDtypeStruct((NC, N_PAD, DH), jnp.float32),
        mesh=mesh,
        compiler_params=pltpu.CompilerParams(needs_layout_passes=False,
                                             use_tc_tiling_on_sc=False),
        scratch_types=[
            pltpu.VMEM((N_PAD,), jnp.float32),
            pltpu.VMEM((CH, K), jnp.int32),
            pltpu.VMEM((CH, K), jnp.int32),
            pltpu.VMEM((CH, K), jnp.float32),
            pltpu.VMEM((K, DH), jnp.float32),
            pltpu.VMEM((NS, ROWS_W), jnp.float32),
            pltpu.VMEM_SHARED((NS, N_PAD), jnp.float32),
            pltpu.VMEM_SHARED((N_PAD,), jnp.float32),
            pltpu.VMEM_SHARED((N_PAD, DH), jnp.float32),
            pltpu.SemaphoreType.DMA,
        ],
    )


_sc_kernel = _make_sc_kernel()


def _tc_body(p0, p1, wci, bci, wli, bli, wcg, bcg, wlg, blg, wco, bco, wlo, blo,
             o_ref, h_ref, c_ref):
    ax = jnp.concatenate([p0[...], p1[...]], axis=1)

    def gate(wc, bc, wl, bl):
        conv = jnp.dot(ax, wc[...], preferred_element_type=jnp.float32) + bc[...]
        return jnp.dot(conv, wl[...], preferred_element_type=jnp.float32) + bl[...]

    i_g = jax.nn.sigmoid(gate(wci, bci, wli, bli))
    g_g = jnp.tanh(gate(wcg, bcg, wlg, blg))
    o_g = jax.nn.sigmoid(gate(wco, bco, wlo, blo))
    cn = i_g * g_g
    o_ref[...] = o_g
    h_ref[...] = o_g * jnp.tanh(cn)
    c_ref[...] = cn


_BLK = 512


def _tc_call(p0, p1, *weights):
    n_blocks = N_PAD // _BLK
    half_spec = pl.BlockSpec((_BLK, DH), lambda i: (i, 0))
    row_spec = pl.BlockSpec((_BLK, D), lambda i: (i, 0))
    w_spec = pl.BlockSpec((D, D), lambda i: (0, 0))
    b_spec = pl.BlockSpec((1, D), lambda i: (0, 0))
    in_specs = [half_spec, half_spec] + [w_spec, b_spec, w_spec, b_spec] * 3
    out_shape = jax.ShapeDtypeStruct((N_PAD, D), jnp.float32)
    return pl.pallas_call(
        _tc_body,
        grid=(n_blocks,),
        in_specs=in_specs,
        out_specs=[row_spec, row_spec, row_spec],
        out_shape=[out_shape, out_shape, out_shape],
    )(p0, p1, *weights)


@jax.jit
def kernel(X, edge_index, edge_weight,
           Wc_i, bc_i, Wl_i, bl_i, Wc_f, bc_f, Wl_f, bl_f,
           Wc_g, bc_g, Wl_g, bl_g, Wc_o, bc_o, Wl_o, bl_o):
    pad_e = E_PAD - E
    src = jnp.pad(edge_index[0], (0, pad_e)).reshape(NG, CH, K)
    dst = jnp.pad(edge_index[1], (0, pad_e)).reshape(NG, CH, K)
    ew = jnp.pad(edge_weight, (0, pad_e)).reshape(NG, CH, K)
    x_pad = jnp.pad(X, ((0, N_PAD - N), (0, 0)))
    x0 = x_pad[:, :DH]
    x1 = x_pad[:, DH:]

    part = _sc_kernel(src, dst, ew, x0, x1)

    weights = []
    for wc, bc, wl, bl in ((Wc_i, bc_i, Wl_i, bl_i),
                           (Wc_g, bc_g, Wl_g, bl_g),
                           (Wc_o, bc_o, Wl_o, bl_o)):
        weights += [wc, bc.reshape(1, D), wl[:D], bl.reshape(1, D)]

    o, h, cn = _tc_call(part[0], part[1], *weights)
    return o[:N], h[:N], cn[:N]


# speedup vs baseline: 25.0057x; 1.3274x over previous
import jax
import jax.numpy as jnp
from jax import lax
from jax.experimental import pallas as pl
from jax.experimental.pallas import tpu as pltpu
from jax.experimental.pallas import tpu_sc as plsc

N = 10000
D = 128
E = 320000

NC = 2
NS = 16
L = 16
DH = D // NC

N_PAD = 10240
ROWS_W = N_PAD // NS
K = 128
CH = 79
EG = CH * K
NG = NC * NS
E_PAD = NG * EG
VEC_IT = EG // L


def _sc_body(src_hbm, dst_hbm, ew_hbm, x0_hbm, x1_hbm, part_hbm,
             degp, idx_src, idx_dst, ewn, rows2, redbuf,
             deg_parts_sh, dis_sh, ax_sh, sem):
    c = lax.axis_index("c")
    s = lax.axis_index("s")

    def _zero(i, _):
        degp[pl.ds(i * L, L)] = jnp.zeros((L,), jnp.float32)
        return 0
    lax.fori_loop(0, N_PAD // L, _zero, 0)

    def _deg_group(g, _):
        pltpu.sync_copy(dst_hbm.at[g], idx_dst)
        pltpu.sync_copy(ew_hbm.at[g], ewn)

        def _dbody(i, _):
            r = i // (K // L)
            k = (i % (K // L)) * L
            di = idx_dst[r, pl.ds(k, L)]
            wv = ewn[r, pl.ds(k, L)]
            plsc.addupdate_scatter(degp, [di], wv)
            return 0
        lax.fori_loop(0, VEC_IT, _dbody, 0)
        return 0
    lax.fori_loop(2 * s, 2 * s + 2, _deg_group, 0)

    pltpu.sync_copy(degp, deg_parts_sh.at[s])
    plsc.subcore_barrier()
    pltpu.sync_copy(deg_parts_sh.at[:, pl.ds(s * ROWS_W, ROWS_W)], redbuf)

    def _red(i, _):
        acc = redbuf[0, pl.ds(i * L, L)]
        for r in range(1, NS):
            acc = acc + redbuf[r, pl.ds(i * L, L)]
        acc = acc + 1.0
        xi = plsc.bitcast(acc, jnp.int32)
        yi = jnp.int32(0x5F3759DF) - lax.shift_right_logical(xi, 1)
        y = plsc.bitcast(yi, jnp.float32)
        for _ in range(3):
            y = y * (1.5 - 0.5 * acc * y * y)
        degp[pl.ds(i * L, L)] = y
        return 0
    lax.fori_loop(0, ROWS_W // L, _red, 0)
    pltpu.sync_copy(degp.at[pl.ds(0, ROWS_W)], dis_sh.at[pl.ds(s * ROWS_W, ROWS_W)])
    plsc.subcore_barrier()

    pltpu.sync_copy(dis_sh, degp)

    def _run_core(xh_hbm):
        def _init_chunk(ch, _):
            base = s * ROWS_W + ch * K
            pltpu.sync_copy(xh_hbm.at[pl.ds(base, K)], rows2.at[0])

            def _rowblk(kb, _):
                disv = degp[pl.ds(base + kb * L, L)]
                scv = disv * disv
                for k in range(L):
                    sc_v = jnp.broadcast_to(scv[k], (L,))
                    row = kb * L + k
                    for j in range(DH // L):
                        rows2[0, row, pl.ds(j * L, L)] = rows2[0, row, pl.ds(j * L, L)] * sc_v
                return 0
            lax.fori_loop(0, K // L, _rowblk, 0)
            pltpu.sync_copy(rows2.at[0], ax_sh.at[pl.ds(base, K)])
            return 0
        lax.fori_loop(0, ROWS_W // K, _init_chunk, 0)
        plsc.subcore_barrier()

        def _group(g, _):
            pltpu.sync_copy(src_hbm.at[g], idx_src)
            pltpu.sync_copy(dst_hbm.at[g], idx_dst)
            pltpu.sync_copy(ew_hbm.at[g], ewn)

            def _norm(i, _):
                r = i // (K // L)
                k = (i % (K // L)) * L
                sv = idx_src[r, pl.ds(k, L)]
                dv = idx_dst[r, pl.ds(k, L)]
                w = ewn[r, pl.ds(k, L)]
                ewn[r, pl.ds(k, L)] = (plsc.load_gather(degp, [sv]) * w
                                       * plsc.load_gather(degp, [dv]))
                return 0
            lax.fori_loop(0, VEC_IT, _norm, 0)

            pltpu.async_copy(xh_hbm.at[idx_src.at[0]], rows2.at[0], sem)

            def _chunk(ci, _):
                b = lax.rem(ci, 2)
                nb = lax.rem(ci + 1, 2)

                @pl.when(ci + 1 < CH)
                def _():
                    pltpu.async_copy(xh_hbm.at[idx_src.at[ci + 1]], rows2.at[nb], sem)

                pltpu.make_async_copy(xh_hbm.at[idx_src.at[ci]], rows2.at[b], sem).wait()

                def _edgeblk(kb, _):
                    nvec = ewn[ci, pl.ds(kb * L, L)]
                    for k in range(L):
                        nv = jnp.broadcast_to(nvec[k], (L,))
                        row = kb * L + k
                        for j in range(DH // L):
                            rows2[b, row, pl.ds(j * L, L)] = rows2[b, row, pl.ds(j * L, L)] * nv
                    return 0
                lax.fori_loop(0, K // L, _edgeblk, 0)
                pltpu.sync_copy(rows2.at[b], ax_sh.at[idx_dst.at[ci]], add=True)
                return 0
            lax.fori_loop(0, CH, _chunk, 0)
            return 0
        lax.fori_loop(2 * s, 2 * s + 2, _group, 0)

    @pl.when(c == 0)
    def _():
        _run_core(x0_hbm)

    @pl.when(c == 1)
    def _():
        _run_core(x1_hbm)

    plsc.subcore_barrier()

    pltpu.sync_copy(ax_sh.at[pl.ds(s * ROWS_W, ROWS_W)],
                    part_hbm.at[c, pl.ds(s * ROWS_W, ROWS_W)])


def _make_sc_kernel():
    mesh = plsc.VectorSubcoreMesh(core_axis_name="c", subcore_axis_name="s",
                                  num_cores=NC, num_subcores=NS)
    return pl.kernel(
        _sc_body,
        out_type=jax.ShapeDtypeStruct((NC, N_PAD, DH), jnp.float32),
        mesh=mesh,
        compiler_params=pltpu.CompilerParams(needs_layout_passes=False,
                                             use_tc_tiling_on_sc=False),
        scratch_types=[
            pltpu.VMEM((N_PAD,), jnp.float32),
            pltpu.VMEM((CH, K), jnp.int32),
            pltpu.VMEM((CH, K), jnp.int32),
            pltpu.VMEM((CH, K), jnp.float32),
            pltpu.VMEM((2, K, DH), jnp.float32),
            pltpu.VMEM((NS, ROWS_W), jnp.float32),
            pltpu.VMEM_SHARED((NS, N_PAD), jnp.float32),
            pltpu.VMEM_SHARED((N_PAD,), jnp.float32),
            pltpu.VMEM_SHARED((N_PAD, DH), jnp.float32),
            pltpu.SemaphoreType.DMA,
        ],
    )


_sc_kernel = _make_sc_kernel()


def _tc_body(p0, p1, wci, bci, wli, bli, wcg, bcg, wlg, blg, wco, bco, wlo, blo,
             o_ref, h_ref, c_ref):
    ax = jnp.concatenate([p0[...], p1[...]], axis=1)

    def gate(wc, bc, wl, bl):
        conv = jnp.dot(ax, wc[...], preferred_element_type=jnp.float32) + bc[...]
        return jnp.dot(conv, wl[...], preferred_element_type=jnp.float32) + bl[...]

    i_g = jax.nn.sigmoid(gate(wci, bci, wli, bli))
    g_g = jnp.tanh(gate(wcg, bcg, wlg, blg))
    o_g = jax.nn.sigmoid(gate(wco, bco, wlo, blo))
    cn = i_g * g_g
    o_ref[...] = o_g
    h_ref[...] = o_g * jnp.tanh(cn)
    c_ref[...] = cn


_BLK = 512


def _tc_call(p0, p1, *weights):
    n_blocks = N_PAD // _BLK
    half_spec = pl.BlockSpec((_BLK, DH), lambda i: (i, 0))
    row_spec = pl.BlockSpec((_BLK, D), lambda i: (i, 0))
    w_spec = pl.BlockSpec((D, D), lambda i: (0, 0))
    b_spec = pl.BlockSpec((1, D), lambda i: (0, 0))
    in_specs = [half_spec, half_spec] + [w_spec, b_spec, w_spec, b_spec] * 3
    out_shape = jax.ShapeDtypeStruct((N_PAD, D), jnp.float32)
    return pl.pallas_call(
        _tc_body,
        grid=(n_blocks,),
        in_specs=in_specs,
        out_specs=[row_spec, row_spec, row_spec],
        out_shape=[out_shape, out_shape, out_shape],
    )(p0, p1, *weights)


@jax.jit
def kernel(X, edge_index, edge_weight,
           Wc_i, bc_i, Wl_i, bl_i, Wc_f, bc_f, Wl_f, bl_f,
           Wc_g, bc_g, Wl_g, bl_g, Wc_o, bc_o, Wl_o, bl_o):
    pad_e = E_PAD - E
    src = jnp.pad(edge_index[0], (0, pad_e)).reshape(NG, CH, K)
    dst = jnp.pad(edge_index[1], (0, pad_e)).reshape(NG, CH, K)
    ew = jnp.pad(edge_weight, (0, pad_e)).reshape(NG, CH, K)
    x_pad = jnp.pad(X, ((0, N_PAD - N), (0, 0)))
    x0 = x_pad[:, :DH]
    x1 = x_pad[:, DH:]

    part = _sc_kernel(src, dst, ew, x0, x1)

    weights = []
    for wc, bc, wl, bl in ((Wc_i, bc_i, Wl_i, bl_i),
                           (Wc_g, bc_g, Wl_g, bl_g),
                           (Wc_o, bc_o, Wl_o, bl_o)):
        weights += [wc, bc.reshape(1, D), wl[:D], bl.reshape(1, D)]

    o, h, cn = _tc_call(part[0], part[1], *weights)
    return o[:N], h[:N], cn[:N]


# speedup vs baseline: 40.0146x; 1.6002x over previous
import jax
import jax.numpy as jnp
from jax import lax
from jax.experimental import pallas as pl
from jax.experimental.pallas import tpu as pltpu
from jax.experimental.pallas import tpu_sc as plsc

N = 10000
D = 128
E = 320000

NC = 2
NS = 16
L = 16
DH = D // NC

N_PAD = 10240
ROWS_W = N_PAD // NS
K = 128
CH = 79
EG = CH * K
NG = NC * NS
E_PAD = NG * EG
VEC_IT = EG // L


def _sc_body(src_hbm, dst_hbm, ew_hbm, x0_hbm, x1_hbm, part_hbm,
             degp, idx_src, idx_dst, ewn, rows2, redbuf,
             deg_parts_sh, dis_sh, ax_sh, sem):
    c = lax.axis_index("c")
    s = lax.axis_index("s")

    def _zero(i, _):
        degp[pl.ds(i * L, L)] = jnp.zeros((L,), jnp.float32)
        return 0
    lax.fori_loop(0, N_PAD // L, _zero, 0)

    def _deg_group(g, _):
        pltpu.sync_copy(dst_hbm.at[g], idx_dst)
        pltpu.sync_copy(ew_hbm.at[g], ewn)

        def _dbody(i, _):
            r = i // (K // L)
            k = (i % (K // L)) * L
            di = idx_dst[r, pl.ds(k, L)]
            wv = ewn[r, pl.ds(k, L)]
            plsc.addupdate_scatter(degp, [di], wv)
            return 0
        lax.fori_loop(0, VEC_IT, _dbody, 0)
        return 0
    lax.fori_loop(2 * s, 2 * s + 2, _deg_group, 0)

    pltpu.sync_copy(degp, deg_parts_sh.at[s])
    plsc.subcore_barrier()
    pltpu.sync_copy(deg_parts_sh.at[:, pl.ds(s * ROWS_W, ROWS_W)], redbuf)

    def _red(i, _):
        acc = redbuf[0, pl.ds(i * L, L)]
        for r in range(1, NS):
            acc = acc + redbuf[r, pl.ds(i * L, L)]
        acc = acc + 1.0
        xi = plsc.bitcast(acc, jnp.int32)
        yi = jnp.int32(0x5F3759DF) - lax.shift_right_logical(xi, 1)
        y = plsc.bitcast(yi, jnp.float32)
        for _ in range(3):
            y = y * (1.5 - 0.5 * acc * y * y)
        degp[pl.ds(i * L, L)] = y
        return 0
    lax.fori_loop(0, ROWS_W // L, _red, 0)
    pltpu.sync_copy(degp.at[pl.ds(0, ROWS_W)], dis_sh.at[pl.ds(s * ROWS_W, ROWS_W)])
    plsc.subcore_barrier()

    pltpu.sync_copy(dis_sh, degp)

    def _run_core(xh_hbm):
        def _init_chunk(ch, _):
            base = s * ROWS_W + ch * K
            pltpu.sync_copy(xh_hbm.at[pl.ds(base, K)], rows2.at[0])

            @plsc.parallel_loop(0, K // L, unroll=2)
            def _rowblk(kb):
                disv = degp[pl.ds(base + kb * L, L)]
                scv = disv * disv
                for k in range(L):
                    sc_v = jnp.broadcast_to(scv[k], (L,))
                    row = kb * L + k
                    vals = [rows2[0, row, pl.ds(j * L, L)] for j in range(DH // L)]
                    for j in range(DH // L):
                        rows2[0, row, pl.ds(j * L, L)] = vals[j] * sc_v
            pltpu.sync_copy(rows2.at[0], ax_sh.at[pl.ds(base, K)])
            return 0
        lax.fori_loop(0, ROWS_W // K, _init_chunk, 0)
        plsc.subcore_barrier()

        def _group(g, _):
            pltpu.sync_copy(src_hbm.at[g], idx_src)
            pltpu.sync_copy(dst_hbm.at[g], idx_dst)
            pltpu.sync_copy(ew_hbm.at[g], ewn)

            def _norm(i, _):
                r = i // (K // L)
                k = (i % (K // L)) * L
                sv = idx_src[r, pl.ds(k, L)]
                dv = idx_dst[r, pl.ds(k, L)]
                w = ewn[r, pl.ds(k, L)]
                ewn[r, pl.ds(k, L)] = (plsc.load_gather(degp, [sv]) * w
                                       * plsc.load_gather(degp, [dv]))
                return 0
            lax.fori_loop(0, VEC_IT, _norm, 0)

            pltpu.async_copy(xh_hbm.at[idx_src.at[0]], rows2.at[0], sem)

            def _chunk(ci, _):
                b = lax.rem(ci, 2)
                nb = lax.rem(ci + 1, 2)

                @pl.when(ci + 1 < CH)
                def _():
                    pltpu.async_copy(xh_hbm.at[idx_src.at[ci + 1]], rows2.at[nb], sem)

                pltpu.make_async_copy(xh_hbm.at[idx_src.at[ci]], rows2.at[b], sem).wait()

                @plsc.parallel_loop(0, K // L, unroll=2)
                def _edgeblk(kb):
                    nvec = ewn[ci, pl.ds(kb * L, L)]
                    for k in range(L):
                        nv = jnp.broadcast_to(nvec[k], (L,))
                        row = kb * L + k
                        vals = [rows2[b, row, pl.ds(j * L, L)] for j in range(DH // L)]
                        for j in range(DH // L):
                            rows2[b, row, pl.ds(j * L, L)] = vals[j] * nv
                pltpu.sync_copy(rows2.at[b], ax_sh.at[idx_dst.at[ci]], add=True)
                return 0
            lax.fori_loop(0, CH, _chunk, 0)
            return 0
        lax.fori_loop(2 * s, 2 * s + 2, _group, 0)

    @pl.when(c == 0)
    def _():
        _run_core(x0_hbm)

    @pl.when(c == 1)
    def _():
        _run_core(x1_hbm)

    plsc.subcore_barrier()

    pltpu.sync_copy(ax_sh.at[pl.ds(s * ROWS_W, ROWS_W)],
                    part_hbm.at[c, pl.ds(s * ROWS_W, ROWS_W)])


def _make_sc_kernel():
    mesh = plsc.VectorSubcoreMesh(core_axis_name="c", subcore_axis_name="s",
                                  num_cores=NC, num_subcores=NS)
    return pl.kernel(
        _sc_body,
        out_type=jax.ShapeDtypeStruct((NC, N_PAD, DH), jnp.float32),
        mesh=mesh,
        compiler_params=pltpu.CompilerParams(needs_layout_passes=False,
                                             use_tc_tiling_on_sc=False),
        scratch_types=[
            pltpu.VMEM((N_PAD,), jnp.float32),
            pltpu.VMEM((CH, K), jnp.int32),
            pltpu.VMEM((CH, K), jnp.int32),
            pltpu.VMEM((CH, K), jnp.float32),
            pltpu.VMEM((2, K, DH), jnp.float32),
            pltpu.VMEM((NS, ROWS_W), jnp.float32),
            pltpu.VMEM_SHARED((NS, N_PAD), jnp.float32),
            pltpu.VMEM_SHARED((N_PAD,), jnp.float32),
            pltpu.VMEM_SHARED((N_PAD, DH), jnp.float32),
            pltpu.SemaphoreType.DMA,
        ],
    )


_sc_kernel = _make_sc_kernel()


def _tc_body(p0, p1, wci, bci, wli, bli, wcg, bcg, wlg, blg, wco, bco, wlo, blo,
             o_ref, h_ref, c_ref):
    ax = jnp.concatenate([p0[...], p1[...]], axis=1)

    def gate(wc, bc, wl, bl):
        conv = jnp.dot(ax, wc[...], preferred_element_type=jnp.float32) + bc[...]
        return jnp.dot(conv, wl[...], preferred_element_type=jnp.float32) + bl[...]

    i_g = jax.nn.sigmoid(gate(wci, bci, wli, bli))
    g_g = jnp.tanh(gate(wcg, bcg, wlg, blg))
    o_g = jax.nn.sigmoid(gate(wco, bco, wlo, blo))
    cn = i_g * g_g
    o_ref[...] = o_g
    h_ref[...] = o_g * jnp.tanh(cn)
    c_ref[...] = cn


_BLK = 512


def _tc_call(p0, p1, *weights):
    n_blocks = N_PAD // _BLK
    half_spec = pl.BlockSpec((_BLK, DH), lambda i: (i, 0))
    row_spec = pl.BlockSpec((_BLK, D), lambda i: (i, 0))
    w_spec = pl.BlockSpec((D, D), lambda i: (0, 0))
    b_spec = pl.BlockSpec((1, D), lambda i: (0, 0))
    in_specs = [half_spec, half_spec] + [w_spec, b_spec, w_spec, b_spec] * 3
    out_shape = jax.ShapeDtypeStruct((N_PAD, D), jnp.float32)
    return pl.pallas_call(
        _tc_body,
        grid=(n_blocks,),
        in_specs=in_specs,
        out_specs=[row_spec, row_spec, row_spec],
        out_shape=[out_shape, out_shape, out_shape],
    )(p0, p1, *weights)


@jax.jit
def kernel(X, edge_index, edge_weight,
           Wc_i, bc_i, Wl_i, bl_i, Wc_f, bc_f, Wl_f, bl_f,
           Wc_g, bc_g, Wl_g, bl_g, Wc_o, bc_o, Wl_o, bl_o):
    pad_e = E_PAD - E
    src = jnp.pad(edge_index[0], (0, pad_e)).reshape(NG, CH, K)
    dst = jnp.pad(edge_index[1], (0, pad_e)).reshape(NG, CH, K)
    ew = jnp.pad(edge_weight, (0, pad_e)).reshape(NG, CH, K)
    x_pad = jnp.pad(X, ((0, N_PAD - N), (0, 0)))
    x0 = x_pad[:, :DH]
    x1 = x_pad[:, DH:]

    part = _sc_kernel(src, dst, ew, x0, x1)

    weights = []
    for wc, bc, wl, bl in ((Wc_i, bc_i, Wl_i, bl_i),
                           (Wc_g, bc_g, Wl_g, bl_g),
                           (Wc_o, bc_o, Wl_o, bl_o)):
        weights += [wc, bc.reshape(1, D), wl[:D], bl.reshape(1, D)]

    o, h, cn = _tc_call(part[0], part[1], *weights)
    return o[:N], h[:N], cn[:N]


# speedup vs baseline: 40.0666x; 1.0013x over previous
import jax
import jax.numpy as jnp
from jax import lax
from jax.experimental import pallas as pl
from jax.experimental.pallas import tpu as pltpu
from jax.experimental.pallas import tpu_sc as plsc

N = 10000
D = 128
E = 320000

NC = 2
NS = 16
L = 16
DH = D // NC

N_PAD = 10240
ROWS_W = N_PAD // NS
K = 128
CH = 79
EG = CH * K
NG = NC * NS
E_PAD = NG * EG
VEC_IT = EG // L


def _sc_body(src_hbm, dst_hbm, ew_hbm, x0_hbm, x1_hbm, part_hbm,
             degp, idx_src, idx_dst, ewn, rows2, redbuf,
             deg_parts_sh, dis_sh, ax_sh, sem, ssem):
    c = lax.axis_index("c")
    s = lax.axis_index("s")

    def _zero(i, _):
        degp[pl.ds(i * L, L)] = jnp.zeros((L,), jnp.float32)
        return 0
    lax.fori_loop(0, N_PAD // L, _zero, 0)

    def _deg_group(g, _):
        pltpu.sync_copy(dst_hbm.at[g], idx_dst)
        pltpu.sync_copy(ew_hbm.at[g], ewn)

        def _dbody(i, _):
            r = i // (K // L)
            k = (i % (K // L)) * L
            di = idx_dst[r, pl.ds(k, L)]
            wv = ewn[r, pl.ds(k, L)]
            plsc.addupdate_scatter(degp, [di], wv)
            return 0
        lax.fori_loop(0, VEC_IT, _dbody, 0)
        return 0
    lax.fori_loop(2 * s, 2 * s + 2, _deg_group, 0)

    pltpu.sync_copy(degp, deg_parts_sh.at[s])
    plsc.subcore_barrier()
    pltpu.sync_copy(deg_parts_sh.at[:, pl.ds(s * ROWS_W, ROWS_W)], redbuf)

    def _red(i, _):
        acc = redbuf[0, pl.ds(i * L, L)]
        for r in range(1, NS):
            acc = acc + redbuf[r, pl.ds(i * L, L)]
        acc = acc + 1.0
        xi = plsc.bitcast(acc, jnp.int32)
        yi = jnp.int32(0x5F3759DF) - lax.shift_right_logical(xi, 1)
        y = plsc.bitcast(yi, jnp.float32)
        for _ in range(3):
            y = y * (1.5 - 0.5 * acc * y * y)
        degp[pl.ds(i * L, L)] = y
        return 0
    lax.fori_loop(0, ROWS_W // L, _red, 0)
    pltpu.sync_copy(degp.at[pl.ds(0, ROWS_W)], dis_sh.at[pl.ds(s * ROWS_W, ROWS_W)])
    plsc.subcore_barrier()

    pltpu.sync_copy(dis_sh, degp)

    def _run_core(xh_hbm):
        def _init_chunk(ch, _):
            base = s * ROWS_W + ch * K
            pltpu.sync_copy(xh_hbm.at[pl.ds(base, K)], rows2.at[0])

            @plsc.parallel_loop(0, K // L, unroll=2)
            def _rowblk(kb):
                disv = degp[pl.ds(base + kb * L, L)]
                scv = disv * disv
                for k in range(L):
                    sc_v = jnp.broadcast_to(scv[k], (L,))
                    row = kb * L + k
                    vals = [rows2[0, row, pl.ds(j * L, L)] for j in range(DH // L)]
                    for j in range(DH // L):
                        rows2[0, row, pl.ds(j * L, L)] = vals[j] * sc_v
            pltpu.sync_copy(rows2.at[0], ax_sh.at[pl.ds(base, K)])
            return 0
        lax.fori_loop(0, ROWS_W // K, _init_chunk, 0)
        plsc.subcore_barrier()

        def _group(g, _):
            pltpu.sync_copy(src_hbm.at[g], idx_src)
            pltpu.sync_copy(dst_hbm.at[g], idx_dst)
            pltpu.sync_copy(ew_hbm.at[g], ewn)

            def _norm(i, _):
                r = i // (K // L)
                k = (i % (K // L)) * L
                sv = idx_src[r, pl.ds(k, L)]
                dv = idx_dst[r, pl.ds(k, L)]
                w = ewn[r, pl.ds(k, L)]
                ewn[r, pl.ds(k, L)] = (plsc.load_gather(degp, [sv]) * w
                                       * plsc.load_gather(degp, [dv]))
                return 0
            lax.fori_loop(0, VEC_IT, _norm, 0)

            pltpu.async_copy(xh_hbm.at[idx_src.at[0]], rows2.at[0], sem)

            def _chunk(ci, _):
                b = lax.rem(ci, 2)
                nb = lax.rem(ci + 1, 2)

                @pl.when(ci >= 1)
                def _():
                    pltpu.make_async_copy(rows2.at[nb],
                                          ax_sh.at[idx_dst.at[ci - 1]], ssem).wait()

                @pl.when(ci + 1 < CH)
                def _():
                    pltpu.async_copy(xh_hbm.at[idx_src.at[ci + 1]], rows2.at[nb], sem)

                pltpu.make_async_copy(xh_hbm.at[idx_src.at[ci]], rows2.at[b], sem).wait()

                @plsc.parallel_loop(0, K // L, unroll=2)
                def _edgeblk(kb):
                    nvec = ewn[ci, pl.ds(kb * L, L)]
                    for k in range(L):
                        nv = jnp.broadcast_to(nvec[k], (L,))
                        row = kb * L + k
                        vals = [rows2[b, row, pl.ds(j * L, L)] for j in range(DH // L)]
                        for j in range(DH // L):
                            rows2[b, row, pl.ds(j * L, L)] = vals[j] * nv
                pltpu.async_copy(rows2.at[b], ax_sh.at[idx_dst.at[ci]], ssem, add=True)
                return 0
            lax.fori_loop(0, CH, _chunk, 0)
            pltpu.make_async_copy(rows2.at[(CH - 1) % 2],
                                  ax_sh.at[idx_dst.at[CH - 1]], ssem).wait()
            return 0
        lax.fori_loop(2 * s, 2 * s + 2, _group, 0)

    @pl.when(c == 0)
    def _():
        _run_core(x0_hbm)

    @pl.when(c == 1)
    def _():
        _run_core(x1_hbm)

    plsc.subcore_barrier()

    pltpu.sync_copy(ax_sh.at[pl.ds(s * ROWS_W, ROWS_W)],
                    part_hbm.at[c, pl.ds(s * ROWS_W, ROWS_W)])


def _make_sc_kernel():
    mesh = plsc.VectorSubcoreMesh(core_axis_name="c", subcore_axis_name="s",
                                  num_cores=NC, num_subcores=NS)
    return pl.kernel(
        _sc_body,
        out_type=jax.ShapeDtypeStruct((NC, N_PAD, DH), jnp.float32),
        mesh=mesh,
        compiler_params=pltpu.CompilerParams(needs_layout_passes=False,
                                             use_tc_tiling_on_sc=False),
        scratch_types=[
            pltpu.VMEM((N_PAD,), jnp.float32),
            pltpu.VMEM((CH, K), jnp.int32),
            pltpu.VMEM((CH, K), jnp.int32),
            pltpu.VMEM((CH, K), jnp.float32),
            pltpu.VMEM((2, K, DH), jnp.float32),
            pltpu.VMEM((NS, ROWS_W), jnp.float32),
            pltpu.VMEM_SHARED((NS, N_PAD), jnp.float32),
            pltpu.VMEM_SHARED((N_PAD,), jnp.float32),
            pltpu.VMEM_SHARED((N_PAD, DH), jnp.float32),
            pltpu.SemaphoreType.DMA,
            pltpu.SemaphoreType.DMA,
        ],
    )


_sc_kernel = _make_sc_kernel()


def _tc_body(p0, p1, wci, bci, wli, bli, wcg, bcg, wlg, blg, wco, bco, wlo, blo,
             o_ref, h_ref, c_ref):
    ax = jnp.concatenate([p0[...], p1[...]], axis=1)

    def gate(wc, bc, wl, bl):
        conv = jnp.dot(ax, wc[...], preferred_element_type=jnp.float32) + bc[...]
        return jnp.dot(conv, wl[...], preferred_element_type=jnp.float32) + bl[...]

    i_g = jax.nn.sigmoid(gate(wci, bci, wli, bli))
    g_g = jnp.tanh(gate(wcg, bcg, wlg, blg))
    o_g = jax.nn.sigmoid(gate(wco, bco, wlo, blo))
    cn = i_g * g_g
    o_ref[...] = o_g
    h_ref[...] = o_g * jnp.tanh(cn)
    c_ref[...] = cn


_BLK = 512


def _tc_call(p0, p1, *weights):
    n_blocks = N_PAD // _BLK
    half_spec = pl.BlockSpec((_BLK, DH), lambda i: (i, 0))
    row_spec = pl.BlockSpec((_BLK, D), lambda i: (i, 0))
    w_spec = pl.BlockSpec((D, D), lambda i: (0, 0))
    b_spec = pl.BlockSpec((1, D), lambda i: (0, 0))
    in_specs = [half_spec, half_spec] + [w_spec, b_spec, w_spec, b_spec] * 3
    out_shape = jax.ShapeDtypeStruct((N_PAD, D), jnp.float32)
    return pl.pallas_call(
        _tc_body,
        grid=(n_blocks,),
        in_specs=in_specs,
        out_specs=[row_spec, row_spec, row_spec],
        out_shape=[out_shape, out_shape, out_shape],
    )(p0, p1, *weights)


@jax.jit
def kernel(X, edge_index, edge_weight,
           Wc_i, bc_i, Wl_i, bl_i, Wc_f, bc_f, Wl_f, bl_f,
           Wc_g, bc_g, Wl_g, bl_g, Wc_o, bc_o, Wl_o, bl_o):
    pad_e = E_PAD - E
    src = jnp.pad(edge_index[0], (0, pad_e)).reshape(NG, CH, K)
    dst = jnp.pad(edge_index[1], (0, pad_e)).reshape(NG, CH, K)
    ew = jnp.pad(edge_weight, (0, pad_e)).reshape(NG, CH, K)
    x_pad = jnp.pad(X, ((0, N_PAD - N), (0, 0)))
    x0 = x_pad[:, :DH]
    x1 = x_pad[:, DH:]

    part = _sc_kernel(src, dst, ew, x0, x1)

    weights = []
    for wc, bc, wl, bl in ((Wc_i, bc_i, Wl_i, bl_i),
                           (Wc_g, bc_g, Wl_g, bl_g),
                           (Wc_o, bc_o, Wl_o, bl_o)):
        weights += [wc, bc.reshape(1, D), wl[:D], bl.reshape(1, D)]

    o, h, cn = _tc_call(part[0], part[1], *weights)
    return o[:N], h[:N], cn[:N]


# speedup vs baseline: 52.8045x; 1.3179x over previous
import jax
import jax.numpy as jnp
from jax import lax
from jax.experimental import pallas as pl
from jax.experimental.pallas import tpu as pltpu
from jax.experimental.pallas import tpu_sc as plsc

N = 10000
D = 128
E = 320000

NC = 2
NS = 16
L = 16
DH = D // NC

N_PAD = 10240
ROWS_W = N_PAD // NS
K = 128
CH = 79
EG = CH * K
NG = NC * NS
E_PAD = NG * EG
VEC_IT = EG // L


def _sc_body(src_hbm, dst_hbm, ew_hbm, x0_hbm, x1_hbm, part_hbm,
             degp, idx_src, idx_dst, ewn, rows_bf, rows_f32, redbuf,
             deg_parts_sh, dis_sh, ax_sh, sem, ssem):
    c = lax.axis_index("c")
    s = lax.axis_index("s")

    def _zero(i, _):
        degp[pl.ds(i * L, L)] = jnp.zeros((L,), jnp.float32)
        return 0
    lax.fori_loop(0, N_PAD // L, _zero, 0)

    def _deg_group(g, _):
        pltpu.sync_copy(dst_hbm.at[g], idx_dst)
        pltpu.sync_copy(ew_hbm.at[g], ewn)

        def _dbody(i, _):
            r = i // (K // L)
            k = (i % (K // L)) * L
            di = idx_dst[r, pl.ds(k, L)]
            wv = ewn[r, pl.ds(k, L)]
            plsc.addupdate_scatter(degp, [di], wv)
            return 0
        lax.fori_loop(0, VEC_IT, _dbody, 0)
        return 0
    lax.fori_loop(2 * s, 2 * s + 2, _deg_group, 0)

    pltpu.sync_copy(degp, deg_parts_sh.at[s])
    plsc.subcore_barrier()
    pltpu.sync_copy(deg_parts_sh.at[:, pl.ds(s * ROWS_W, ROWS_W)], redbuf)

    def _red(i, _):
        acc = redbuf[0, pl.ds(i * L, L)]
        for r in range(1, NS):
            acc = acc + redbuf[r, pl.ds(i * L, L)]
        acc = acc + 1.0
        xi = plsc.bitcast(acc, jnp.int32)
        yi = jnp.int32(0x5F3759DF) - lax.shift_right_logical(xi, 1)
        y = plsc.bitcast(yi, jnp.float32)
        for _ in range(3):
            y = y * (1.5 - 0.5 * acc * y * y)
        degp[pl.ds(i * L, L)] = y
        return 0
    lax.fori_loop(0, ROWS_W // L, _red, 0)
    pltpu.sync_copy(degp.at[pl.ds(0, ROWS_W)], dis_sh.at[pl.ds(s * ROWS_W, ROWS_W)])
    plsc.subcore_barrier()

    pltpu.sync_copy(dis_sh, degp)

    iot = lax.iota(jnp.int32, L)
    idx_even = iot * 2
    idx_odd = idx_even + 1

    def _scale_block(bf_buf, bi, f32_view, kb, scale_vec):
        for k in range(L):
            nv = jnp.broadcast_to(scale_vec[k], (L,))
            row = kb * L + k
            rsp = jnp.broadcast_to(row, (L,))
            for g in range(DH // (2 * L)):
                v = bf_buf[bi, row, pl.ds(g * 2 * L, 2 * L)]
                a, b2 = plsc.unpack(v, format=plsc.PackFormat.INTERLEAVED,
                                    preferred_element_type=jnp.float32)
                cb = g * 2 * L
                plsc.store_scatter(f32_view, [rsp, cb + idx_even], a * nv)
                plsc.store_scatter(f32_view, [rsp, cb + idx_odd], b2 * nv)

    def _run_core(xh_hbm):
        def _init_chunk(ch, _):
            base = s * ROWS_W + ch * K
            pltpu.sync_copy(xh_hbm.at[pl.ds(base, K)], rows_bf.at[0])

            @plsc.parallel_loop(0, K // L, unroll=2)
            def _rowblk(kb):
                disv = degp[pl.ds(base + kb * L, L)]
                _scale_block(rows_bf, 0, rows_f32.at[0], kb, disv * disv)
            pltpu.sync_copy(rows_f32.at[0], ax_sh.at[pl.ds(base, K)])
            return 0
        lax.fori_loop(0, ROWS_W // K, _init_chunk, 0)
        plsc.subcore_barrier()

        def _group(g, _):
            pltpu.sync_copy(src_hbm.at[g], idx_src)
            pltpu.sync_copy(dst_hbm.at[g], idx_dst)
            pltpu.sync_copy(ew_hbm.at[g], ewn)

            def _norm(i, _):
                r = i // (K // L)
                k = (i % (K // L)) * L
                sv = idx_src[r, pl.ds(k, L)]
                dv = idx_dst[r, pl.ds(k, L)]
                w = ewn[r, pl.ds(k, L)]
                ewn[r, pl.ds(k, L)] = (plsc.load_gather(degp, [sv]) * w
                                       * plsc.load_gather(degp, [dv]))
                return 0
            lax.fori_loop(0, VEC_IT, _norm, 0)

            pltpu.async_copy(xh_hbm.at[idx_src.at[0]], rows_bf.at[0], sem)

            def _chunk(ci, _):
                b = lax.rem(ci, 2)
                nb = lax.rem(ci + 1, 2)

                @pl.when(ci >= 2)
                def _():
                    pltpu.make_async_copy(rows_f32.at[b],
                                          ax_sh.at[idx_dst.at[ci - 2]], ssem).wait()

                @pl.when(ci + 1 < CH)
                def _():
                    pltpu.async_copy(xh_hbm.at[idx_src.at[ci + 1]],
                                     rows_bf.at[nb], sem)

                pltpu.make_async_copy(xh_hbm.at[idx_src.at[ci]], rows_bf.at[b], sem).wait()

                @plsc.parallel_loop(0, K // L, unroll=2)
                def _edgeblk(kb):
                    nvec = ewn[ci, pl.ds(kb * L, L)]
                    _scale_block(rows_bf, b, rows_f32.at[b], kb, nvec)
                pltpu.async_copy(rows_f32.at[b], ax_sh.at[idx_dst.at[ci]], ssem, add=True)
                return 0
            lax.fori_loop(0, CH, _chunk, 0)
            pltpu.make_async_copy(rows_f32.at[(CH - 2) % 2],
                                  ax_sh.at[idx_dst.at[CH - 2]], ssem).wait()
            pltpu.make_async_copy(rows_f32.at[(CH - 1) % 2],
                                  ax_sh.at[idx_dst.at[CH - 1]], ssem).wait()
            return 0
        lax.fori_loop(2 * s, 2 * s + 2, _group, 0)

    @pl.when(c == 0)
    def _():
        _run_core(x0_hbm)

    @pl.when(c == 1)
    def _():
        _run_core(x1_hbm)

    plsc.subcore_barrier()

    pltpu.sync_copy(ax_sh.at[pl.ds(s * ROWS_W, ROWS_W)],
                    part_hbm.at[c, pl.ds(s * ROWS_W, ROWS_W)])


def _make_sc_kernel():
    mesh = plsc.VectorSubcoreMesh(core_axis_name="c", subcore_axis_name="s",
                                  num_cores=NC, num_subcores=NS)
    return pl.kernel(
        _sc_body,
        out_type=jax.ShapeDtypeStruct((NC, N_PAD, DH), jnp.float32),
        mesh=mesh,
        compiler_params=pltpu.CompilerParams(needs_layout_passes=False,
                                             use_tc_tiling_on_sc=False),
        scratch_types=[
            pltpu.VMEM((N_PAD,), jnp.float32),
            pltpu.VMEM((CH, K), jnp.int32),
            pltpu.VMEM((CH, K), jnp.int32),
            pltpu.VMEM((CH, K), jnp.float32),
            pltpu.VMEM((2, K, DH), jnp.bfloat16),
            pltpu.VMEM((2, K, DH), jnp.float32),
            pltpu.VMEM((NS, ROWS_W), jnp.float32),
            pltpu.VMEM_SHARED((NS, N_PAD), jnp.float32),
            pltpu.VMEM_SHARED((N_PAD,), jnp.float32),
            pltpu.VMEM_SHARED((N_PAD, DH), jnp.float32),
            pltpu.SemaphoreType.DMA,
            pltpu.SemaphoreType.DMA,
        ],
    )


_sc_kernel = _make_sc_kernel()


def _tc_body(p0, p1, wci, bci, wli, bli, wcg, bcg, wlg, blg, wco, bco, wlo, blo,
             o_ref, h_ref, c_ref):
    ax = jnp.concatenate([p0[...], p1[...]], axis=1)

    def gate(wc, bc, wl, bl):
        conv = jnp.dot(ax, wc[...], preferred_element_type=jnp.float32) + bc[...]
        return jnp.dot(conv, wl[...], preferred_element_type=jnp.float32) + bl[...]

    i_g = jax.nn.sigmoid(gate(wci, bci, wli, bli))
    g_g = jnp.tanh(gate(wcg, bcg, wlg, blg))
    o_g = jax.nn.sigmoid(gate(wco, bco, wlo, blo))
    cn = i_g * g_g
    o_ref[...] = o_g
    h_ref[...] = o_g * jnp.tanh(cn)
    c_ref[...] = cn


_BLK = 512


def _tc_call(p0, p1, *weights):
    n_blocks = N_PAD // _BLK
    half_spec = pl.BlockSpec((_BLK, DH), lambda i: (i, 0))
    row_spec = pl.BlockSpec((_BLK, D), lambda i: (i, 0))
    w_spec = pl.BlockSpec((D, D), lambda i: (0, 0))
    b_spec = pl.BlockSpec((1, D), lambda i: (0, 0))
    in_specs = [half_spec, half_spec] + [w_spec, b_spec, w_spec, b_spec] * 3
    out_shape = jax.ShapeDtypeStruct((N, D), jnp.float32)
    return pl.pallas_call(
        _tc_body,
        grid=(n_blocks,),
        in_specs=in_specs,
        out_specs=[row_spec, row_spec, row_spec],
        out_shape=[out_shape, out_shape, out_shape],
    )(p0, p1, *weights)


@jax.jit
def kernel(X, edge_index, edge_weight,
           Wc_i, bc_i, Wl_i, bl_i, Wc_f, bc_f, Wl_f, bl_f,
           Wc_g, bc_g, Wl_g, bl_g, Wc_o, bc_o, Wl_o, bl_o):
    pad_e = E_PAD - E
    src = jnp.pad(edge_index[0], (0, pad_e)).reshape(NG, CH, K)
    dst = jnp.pad(edge_index[1], (0, pad_e)).reshape(NG, CH, K)
    ew = jnp.pad(edge_weight, (0, pad_e)).reshape(NG, CH, K)
    x_pad = jnp.pad(X, ((0, N_PAD - N), (0, 0))).astype(jnp.bfloat16)
    x0 = x_pad[:, :DH]
    x1 = x_pad[:, DH:]

    part = _sc_kernel(src, dst, ew, x0, x1)

    weights = []
    for wc, bc, wl, bl in ((Wc_i, bc_i, Wl_i, bl_i),
                           (Wc_g, bc_g, Wl_g, bl_g),
                           (Wc_o, bc_o, Wl_o, bl_o)):
        weights += [wc, bc.reshape(1, D), wl[:D], bl.reshape(1, D)]

    o, h, cn = _tc_call(part[0], part[1], *weights)
    return o, h, cn


# speedup vs baseline: 54.1730x; 1.0259x over previous
import jax
import jax.numpy as jnp
from jax import lax
from jax.experimental import pallas as pl
from jax.experimental.pallas import tpu as pltpu
from jax.experimental.pallas import tpu_sc as plsc

N = 10000
D = 128
E = 320000

NC = 2
NS = 16
L = 16
DH = D // NC

N_PAD = 10240
ROWS_W = N_PAD // NS
K = 128
CH = 79
EG = CH * K
NG = NC * NS
E_PAD = NG * EG
VEC_IT = EG // L


def _sc_body(src_hbm, dst_hbm, ew_hbm, x0_hbm, x1_hbm, part_hbm,
             degp, idx_src, idx_dst, ewn, rows_bf, rows_f32, redbuf,
             deg_parts_sh, dis_sh, ax_sh, sem, ssem):
    c = lax.axis_index("c")
    s = lax.axis_index("s")

    @plsc.parallel_loop(0, N_PAD // L, unroll=4)
    def _zero(i):
        degp[pl.ds(i * L, L)] = jnp.zeros((L,), jnp.float32)

    def _deg_group(g, _):
        pltpu.sync_copy(dst_hbm.at[g], idx_dst)
        pltpu.sync_copy(ew_hbm.at[g], ewn)

        def _dbody(i, _):
            r = i // (K // L)
            k = (i % (K // L)) * L
            di = idx_dst[r, pl.ds(k, L)]
            wv = ewn[r, pl.ds(k, L)]
            plsc.addupdate_scatter(degp, [di], wv)
            return 0
        lax.fori_loop(0, VEC_IT, _dbody, 0)
        return 0
    lax.fori_loop(2 * s, 2 * s + 2, _deg_group, 0)

    pltpu.sync_copy(degp, deg_parts_sh.at[s])
    plsc.subcore_barrier()
    pltpu.sync_copy(deg_parts_sh.at[:, pl.ds(s * ROWS_W, ROWS_W)], redbuf)

    @plsc.parallel_loop(0, ROWS_W // L, unroll=2)
    def _red(i):
        acc = redbuf[0, pl.ds(i * L, L)]
        for r in range(1, NS):
            acc = acc + redbuf[r, pl.ds(i * L, L)]
        acc = acc + 1.0
        xi = plsc.bitcast(acc, jnp.int32)
        yi = jnp.int32(0x5F3759DF) - lax.shift_right_logical(xi, 1)
        y = plsc.bitcast(yi, jnp.float32)
        for _ in range(3):
            y = y * (1.5 - 0.5 * acc * y * y)
        degp[pl.ds(i * L, L)] = y
    pltpu.sync_copy(degp.at[pl.ds(0, ROWS_W)], dis_sh.at[pl.ds(s * ROWS_W, ROWS_W)])
    plsc.subcore_barrier()

    pltpu.sync_copy(dis_sh, degp)

    iot = lax.iota(jnp.int32, L)
    idx_even = iot * 2
    idx_odd = idx_even + 1

    def _scale_block(bf_buf, bi, f32_view, kb, scale_vec):
        for k in range(L):
            nv = jnp.broadcast_to(scale_vec[k], (L,))
            row = kb * L + k
            rsp = jnp.broadcast_to(row, (L,))
            for g in range(DH // (2 * L)):
                v = bf_buf[bi, row, pl.ds(g * 2 * L, 2 * L)]
                a, b2 = plsc.unpack(v, format=plsc.PackFormat.INTERLEAVED,
                                    preferred_element_type=jnp.float32)
                cb = g * 2 * L
                plsc.store_scatter(f32_view, [rsp, cb + idx_even], a * nv)
                plsc.store_scatter(f32_view, [rsp, cb + idx_odd], b2 * nv)

    def _run_core(xh_hbm):
        def _init_chunk(ch, _):
            base = s * ROWS_W + ch * K
            pltpu.sync_copy(xh_hbm.at[pl.ds(base, K)], rows_bf.at[0])

            @plsc.parallel_loop(0, K // L, unroll=2)
            def _rowblk(kb):
                disv = degp[pl.ds(base + kb * L, L)]
                _scale_block(rows_bf, 0, rows_f32.at[0], kb, disv * disv)
            pltpu.sync_copy(rows_f32.at[0], ax_sh.at[pl.ds(base, K)])
            return 0
        lax.fori_loop(0, ROWS_W // K, _init_chunk, 0)
        plsc.subcore_barrier()

        def _group(g, _):
            pltpu.sync_copy(src_hbm.at[g], idx_src)
            pltpu.sync_copy(dst_hbm.at[g], idx_dst)
            pltpu.sync_copy(ew_hbm.at[g], ewn)

            @plsc.parallel_loop(0, VEC_IT, unroll=2)
            def _norm(i):
                r = i // (K // L)
                k = (i % (K // L)) * L
                sv = idx_src[r, pl.ds(k, L)]
                dv = idx_dst[r, pl.ds(k, L)]
                w = ewn[r, pl.ds(k, L)]
                ewn[r, pl.ds(k, L)] = (plsc.load_gather(degp, [sv]) * w
                                       * plsc.load_gather(degp, [dv]))

            pltpu.async_copy(xh_hbm.at[idx_src.at[0]], rows_bf.at[0], sem)

            def _chunk(ci, _):
                b = lax.rem(ci, 2)
                nb = lax.rem(ci + 1, 2)

                @pl.when(ci >= 2)
                def _():
                    pltpu.make_async_copy(rows_f32.at[b],
                                          ax_sh.at[idx_dst.at[ci - 2]], ssem).wait()

                @pl.when(ci + 1 < CH)
                def _():
                    pltpu.async_copy(xh_hbm.at[idx_src.at[ci + 1]],
                                     rows_bf.at[nb], sem)

                pltpu.make_async_copy(xh_hbm.at[idx_src.at[ci]], rows_bf.at[b], sem).wait()

                @plsc.parallel_loop(0, K // L, unroll=2)
                def _edgeblk(kb):
                    nvec = ewn[ci, pl.ds(kb * L, L)]
                    _scale_block(rows_bf, b, rows_f32.at[b], kb, nvec)
                pltpu.async_copy(rows_f32.at[b], ax_sh.at[idx_dst.at[ci]], ssem, add=True)
                return 0
            lax.fori_loop(0, CH, _chunk, 0)
            pltpu.make_async_copy(rows_f32.at[(CH - 2) % 2],
                                  ax_sh.at[idx_dst.at[CH - 2]], ssem).wait()
            pltpu.make_async_copy(rows_f32.at[(CH - 1) % 2],
                                  ax_sh.at[idx_dst.at[CH - 1]], ssem).wait()
            return 0
        lax.fori_loop(2 * s, 2 * s + 2, _group, 0)

    @pl.when(c == 0)
    def _():
        _run_core(x0_hbm)

    @pl.when(c == 1)
    def _():
        _run_core(x1_hbm)

    plsc.subcore_barrier()

    pltpu.sync_copy(ax_sh.at[pl.ds(s * ROWS_W, ROWS_W)],
                    part_hbm.at[c, pl.ds(s * ROWS_W, ROWS_W)])


def _make_sc_kernel():
    mesh = plsc.VectorSubcoreMesh(core_axis_name="c", subcore_axis_name="s",
                                  num_cores=NC, num_subcores=NS)
    return pl.kernel(
        _sc_body,
        out_type=jax.ShapeDtypeStruct((NC, N_PAD, DH), jnp.float32),
        mesh=mesh,
        compiler_params=pltpu.CompilerParams(needs_layout_passes=False,
                                             use_tc_tiling_on_sc=False),
        scratch_types=[
            pltpu.VMEM((N_PAD,), jnp.float32),
            pltpu.VMEM((CH, K), jnp.int32),
            pltpu.VMEM((CH, K), jnp.int32),
            pltpu.VMEM((CH, K), jnp.float32),
            pltpu.VMEM((2, K, DH), jnp.bfloat16),
            pltpu.VMEM((2, K, DH), jnp.float32),
            pltpu.VMEM((NS, ROWS_W), jnp.float32),
            pltpu.VMEM_SHARED((NS, N_PAD), jnp.float32),
            pltpu.VMEM_SHARED((N_PAD,), jnp.float32),
            pltpu.VMEM_SHARED((N_PAD, DH), jnp.float32),
            pltpu.SemaphoreType.DMA,
            pltpu.SemaphoreType.DMA,
        ],
    )


_sc_kernel = _make_sc_kernel()


def _tc_body(p0, p1, wci, bci, wli, bli, wcg, bcg, wlg, blg, wco, bco, wlo, blo,
             o_ref, h_ref, c_ref):
    ax = jnp.concatenate([p0[...], p1[...]], axis=1)

    def gate(wc, bc, wl, bl):
        conv = jnp.dot(ax, wc[...], preferred_element_type=jnp.float32) + bc[...]
        return jnp.dot(conv, wl[...], preferred_element_type=jnp.float32) + bl[...]

    i_g = jax.nn.sigmoid(gate(wci, bci, wli, bli))
    g_g = jnp.tanh(gate(wcg, bcg, wlg, blg))
    o_g = jax.nn.sigmoid(gate(wco, bco, wlo, blo))
    cn = i_g * g_g
    o_ref[...] = o_g
    h_ref[...] = o_g * jnp.tanh(cn)
    c_ref[...] = cn


_BLK = 512


def _tc_call(p0, p1, *weights):
    n_blocks = N_PAD // _BLK
    half_spec = pl.BlockSpec((_BLK, DH), lambda i: (i, 0))
    row_spec = pl.BlockSpec((_BLK, D), lambda i: (i, 0))
    w_spec = pl.BlockSpec((D, D), lambda i: (0, 0))
    b_spec = pl.BlockSpec((1, D), lambda i: (0, 0))
    in_specs = [half_spec, half_spec] + [w_spec, b_spec, w_spec, b_spec] * 3
    out_shape = jax.ShapeDtypeStruct((N, D), jnp.float32)
    return pl.pallas_call(
        _tc_body,
        grid=(n_blocks,),
        in_specs=in_specs,
        out_specs=[row_spec, row_spec, row_spec],
        out_shape=[out_shape, out_shape, out_shape],
    )(p0, p1, *weights)


@jax.jit
def kernel(X, edge_index, edge_weight,
           Wc_i, bc_i, Wl_i, bl_i, Wc_f, bc_f, Wl_f, bl_f,
           Wc_g, bc_g, Wl_g, bl_g, Wc_o, bc_o, Wl_o, bl_o):
    pad_e = E_PAD - E
    src = jnp.pad(edge_index[0], (0, pad_e)).reshape(NG, CH, K)
    dst = jnp.pad(edge_index[1], (0, pad_e)).reshape(NG, CH, K)
    ew = jnp.pad(edge_weight, (0, pad_e)).reshape(NG, CH, K)
    x_pad = jnp.pad(X, ((0, N_PAD - N), (0, 0))).astype(jnp.bfloat16)
    x0 = x_pad[:, :DH]
    x1 = x_pad[:, DH:]

    part = _sc_kernel(src, dst, ew, x0, x1)

    weights = []
    for wc, bc, wl, bl in ((Wc_i, bc_i, Wl_i, bl_i),
                           (Wc_g, bc_g, Wl_g, bl_g),
                           (Wc_o, bc_o, Wl_o, bl_o)):
        weights += [wc, bc.reshape(1, D), wl[:D], bl.reshape(1, D)]

    o, h, cn = _tc_call(part[0], part[1], *weights)
    return o, h, cn
